# baseline (speedup 1.0000x reference)
import functools

import jax
import jax.numpy as jnp
from jax import lax
from jax.experimental import pallas as pl
from jax.experimental.pallas import tpu as pltpu
from jax.experimental.pallas import tpu_sc as plsc

CH = 128
GRP = 4
LANES = 16
N_TILES = 32


def _edge_weight_body(n_dist, n_phi, params_ref, dd_ref, dp_ref, c_ref):
    dd = dd_ref[...]
    dp = dp_ref[...]
    sigma = params_ref[n_dist + n_phi]
    kappa = params_ref[n_dist + n_phi + 1]
    wd = jnp.zeros_like(dd)
    for k in range(n_dist):
        wd = wd + jnp.exp(-0.5 * ((dd - params_ref[k]) / sigma) ** 2)
    wp = jnp.zeros_like(dp)
    for j in range(n_phi):
        wp = wp + jnp.exp(kappa * jnp.cos(dp - params_ref[n_dist + j]))
    s = wd * wp
    c_ref[...] = s / (s + 1e-9)


def _sc_body(n_pad, chunks_per_tile, d,
             src_hbm, dst_hbm, c_hbm, x_hbm, feat_hbm, deg_hbm,
             src_g0, src_g1, dst_g0, dst_g1, c_g0, c_g1,
             rows0, rows1, ones_b, feat_acc, deg_acc,
             gsem0, gsem1, ssem0, ssem1, psem0, psem1):
    ci = lax.axis_index("c")
    si = lax.axis_index("s")
    rows_per_tile = n_pad // 16
    n_groups = chunks_per_tile // GRP
    rows_b = (rows0, rows1)
    src_g = (src_g0, src_g1)
    dst_g = (dst_g0, dst_g1)
    c_g = (c_g0, c_g1)
    gsem = (gsem0, gsem1)
    ssem = (ssem0, ssem1)
    psem = (psem0, psem1)

    zero16 = jnp.zeros((LANES,), jnp.float32)

    def zrow(r, carry):
        for k in range(d // LANES):
            rows0[r, pl.ds(k * LANES, LANES)] = zero16
        ones_b[r, pl.ds(0, LANES)] = zero16
        return carry

    lax.fori_loop(0, CH, zrow, 0)
    for j in range(rows_per_tile // CH):
        r0 = si * rows_per_tile + j * CH
        pltpu.sync_copy(rows0, feat_acc.at[pl.ds(r0, CH)])
        pltpu.sync_copy(ones_b, deg_acc.at[pl.ds(r0, CH)])

    one16 = jnp.ones((LANES,), jnp.float32)

    def orow(r, carry):
        ones_b[r, pl.ds(0, LANES)] = one16
        return carry

    lax.fori_loop(0, CH, orow, 0)

    base = (ci * 16 + si) * chunks_per_tile
    pltpu.sync_copy(src_hbm.at[pl.ds(base, GRP)], src_g[0])
    pltpu.sync_copy(dst_hbm.at[pl.ds(base, GRP)], dst_g[0])
    pltpu.sync_copy(c_hbm.at[pl.ds(base, GRP)], c_g[0])
    plsc.subcore_barrier()

    pltpu.async_copy(x_hbm.at[src_g[0].at[0]], rows_b[0], gsem[0])

    def group_pair(gg, carry):
        for gb in range(2):
            g = gg * 2 + gb
            for cb in range(GRP):
                b = cb % 2
                t = g * GRP + cb

                prev_dst = dst_g[1 - gb].at[GRP - 1] if cb == 0 \
                    else dst_g[gb].at[cb - 1]

                @pl.when(t >= 1)
                def _drain():
                    pltpu.make_async_copy(
                        rows_b[1 - b], feat_acc.at[prev_dst], ssem[1 - b]
                    ).wait()
                    pltpu.make_async_copy(
                        ones_b, deg_acc.at[prev_dst], ssem[1 - b]).wait()

                if cb == 0:
                    @pl.when(g + 1 < n_groups)
                    def _prefetch():
                        nb = base + (g + 1) * GRP
                        pltpu.async_copy(src_hbm.at[pl.ds(nb, GRP)],
                                         src_g[1 - gb], psem[1 - gb])
                        pltpu.async_copy(dst_hbm.at[pl.ds(nb, GRP)],
                                         dst_g[1 - gb], psem[1 - gb])
                        pltpu.async_copy(c_hbm.at[pl.ds(nb, GRP)],
                                         c_g[1 - gb], psem[1 - gb])

                if cb < GRP - 1:
                    @pl.when(t + 1 < GRP * n_groups)
                    def _gather_next():
                        pltpu.async_copy(x_hbm.at[src_g[gb].at[cb + 1]],
                                         rows_b[1 - b], gsem[1 - b])
                else:
                    @pl.when(g + 1 < n_groups)
                    def _gather_next_group():
                        nbuf = 1 - gb
                        nb = base + (g + 1) * GRP
                        pltpu.make_async_copy(src_hbm.at[pl.ds(nb, GRP)],
                                              src_g[nbuf], psem[nbuf]).wait()
                        pltpu.make_async_copy(dst_hbm.at[pl.ds(nb, GRP)],
                                              dst_g[nbuf], psem[nbuf]).wait()
                        pltpu.make_async_copy(c_hbm.at[pl.ds(nb, GRP)],
                                              c_g[nbuf], psem[nbuf]).wait()
                        pltpu.async_copy(x_hbm.at[src_g[nbuf].at[0]],
                                         rows_b[1 - b], gsem[1 - b])

                pltpu.make_async_copy(x_hbm.at[src_g[gb].at[cb]],
                                      rows_b[b], gsem[b]).wait()

                def srow(r, inner):
                    cbv = plsc.load_gather(
                        c_g[gb],
                        [jnp.full((LANES,), cb, jnp.int32),
                         jnp.full((LANES,), r, jnp.int32)])
                    for k in range(d // LANES):
                        sl = pl.ds(k * LANES, LANES)
                        rows_b[b][r, sl] = rows_b[b][r, sl] * cbv
                    return inner

                lax.fori_loop(0, CH, srow, 0)

                cur_dst = dst_g[gb].at[cb]
                pltpu.async_copy(rows_b[b], feat_acc.at[cur_dst],
                                 ssem[b], add=True)
                pltpu.async_copy(ones_b, deg_acc.at[cur_dst],
                                 ssem[b], add=True)
        return carry

    lax.fori_loop(0, n_groups // 2, group_pair, 0)

    last_b = (chunks_per_tile - 1) % 2
    last_dst = dst_g[(n_groups - 1) % 2].at[GRP - 1]
    pltpu.make_async_copy(rows_b[last_b], feat_acc.at[last_dst],
                          ssem[last_b]).wait()
    pltpu.make_async_copy(ones_b, deg_acc.at[last_dst], ssem[last_b]).wait()
    plsc.subcore_barrier()

    r0 = si * rows_per_tile
    pltpu.sync_copy(feat_acc.at[pl.ds(r0, rows_per_tile)],
                    feat_hbm.at[pl.ds(ci * n_pad + r0, rows_per_tile)])
    pltpu.sync_copy(deg_acc.at[pl.ds(r0, rows_per_tile)],
                    deg_hbm.at[pl.ds(ci * n_pad + r0, rows_per_tile)])


def _combine_body(n_pad, d, f_ref, g_ref, o_ref):
    s = f_ref[:n_pad, :] + f_ref[n_pad:, :]
    deg = g_ref[:n_pad, :1] + g_ref[n_pad:, :1]
    o_ref[...] = s / (4.0 * (deg + 1e-9))


def kernel(x, edge_index, d_dists, d_phi, dists, sigma, phi, kappa):
    n, d = x.shape
    e = edge_index.shape[1]
    n_dist = dists.shape[0]
    n_phi = phi.shape[0]
    e_pad = -(-e // (N_TILES * CH)) * (N_TILES * CH)
    n_pad = -(-(n + 8) // (16 * CH)) * (16 * CH)
    chunks_per_tile = e_pad // (N_TILES * CH)

    params = jnp.concatenate(
        [dists, phi, sigma[None], kappa[None]]).astype(jnp.float32)
    rows_a = e // CH
    c2d = pl.pallas_call(
        functools.partial(_edge_weight_body, n_dist, n_phi),
        out_shape=jax.ShapeDtypeStruct((rows_a, CH), jnp.float32),
        in_specs=[
            pl.BlockSpec(memory_space=pltpu.SMEM),
            pl.BlockSpec(memory_space=pltpu.VMEM),
            pl.BlockSpec(memory_space=pltpu.VMEM),
        ],
        out_specs=pl.BlockSpec(memory_space=pltpu.VMEM),
    )(params, d_dists.reshape(rows_a, CH), d_phi.reshape(rows_a, CH))
    c = c2d.reshape(e)

    pad = e_pad - e
    n_chunks = e_pad // CH
    src_p = jnp.concatenate(
        [edge_index[0], jnp.zeros((pad,), jnp.int32)]).reshape(n_chunks, CH)
    dst_p = jnp.concatenate(
        [edge_index[1],
         jnp.full((pad,), n_pad - 8, jnp.int32)]).reshape(n_chunks, CH)
    c_p = jnp.concatenate(
        [c, jnp.zeros((pad,), jnp.float32)]).reshape(n_chunks, CH)

    mesh = plsc.VectorSubcoreMesh(core_axis_name="c", subcore_axis_name="s")
    sc_fn = pl.kernel(
        functools.partial(_sc_body, n_pad, chunks_per_tile, d),
        mesh=mesh,
        compiler_params=pltpu.CompilerParams(use_tc_tiling_on_sc=False,
                                             needs_layout_passes=False),
        out_type=[
            jax.ShapeDtypeStruct((2 * n_pad, d), jnp.float32),
            jax.ShapeDtypeStruct((2 * n_pad, LANES), jnp.float32),
        ],
        scratch_types=[
            pltpu.VMEM((GRP, CH), jnp.int32),
            pltpu.VMEM((GRP, CH), jnp.int32),
            pltpu.VMEM((GRP, CH), jnp.int32),
            pltpu.VMEM((GRP, CH), jnp.int32),
            pltpu.VMEM((GRP, CH), jnp.float32),
            pltpu.VMEM((GRP, CH), jnp.float32),
            pltpu.VMEM((CH, d), jnp.float32),
            pltpu.VMEM((CH, d), jnp.float32),
            pltpu.VMEM((CH, LANES), jnp.float32),
            pltpu.VMEM_SHARED((n_pad, d), jnp.float32),
            pltpu.VMEM_SHARED((n_pad, LANES), jnp.float32),
            pltpu.SemaphoreType.DMA,
            pltpu.SemaphoreType.DMA,
            pltpu.SemaphoreType.DMA,
            pltpu.SemaphoreType.DMA,
            pltpu.SemaphoreType.DMA,
            pltpu.SemaphoreType.DMA,
        ],
    )
    feat_p, deg_p = sc_fn(src_p, dst_p, c_p, x)

    out_pad = pl.pallas_call(
        functools.partial(_combine_body, n_pad, d),
        out_shape=jax.ShapeDtypeStruct((n_pad, d), jnp.float32),
        in_specs=[pl.BlockSpec(memory_space=pltpu.VMEM),
                  pl.BlockSpec(memory_space=pltpu.VMEM)],
        out_specs=pl.BlockSpec(memory_space=pltpu.VMEM),
    )(feat_p, deg_p)
    return out_pad[:n, :]


# speedup vs baseline: 1.3672x; 1.3672x over previous
import functools

import jax
import jax.numpy as jnp
from jax import lax
from jax.experimental import pallas as pl
from jax.experimental.pallas import tpu as pltpu
from jax.experimental.pallas import tpu_sc as plsc

CH = 128
GRP = 4
LANES = 16
N_TILES = 32


def _edge_weight_body(n_dist, n_phi, params_ref, dd_ref, dp_ref, c_ref):
    dd = dd_ref[...]
    dp = dp_ref[...]
    sigma = params_ref[n_dist + n_phi]
    kappa = params_ref[n_dist + n_phi + 1]
    wd = jnp.zeros_like(dd)
    for k in range(n_dist):
        wd = wd + jnp.exp(-0.5 * ((dd - params_ref[k]) / sigma) ** 2)
    wp = jnp.zeros_like(dp)
    for j in range(n_phi):
        wp = wp + jnp.exp(kappa * jnp.cos(dp - params_ref[n_dist + j]))
    s = wd * wp
    c_ref[...] = s / (s + 1e-9)


def _sc_body(n_pad, chunks_per_tile, d,
             src_hbm, dst_hbm, c_hbm, x_hbm, feat_hbm, deg_hbm,
             src_g0, src_g1, dst_g0, dst_g1, c_g0, c_g1,
             rows0, rows1, fout0, sidx, ones_b, feat_acc, deg_acc,
             gsem0, gsem1, ssem0, psem0, psem1):
    ci = lax.axis_index("c")
    si = lax.axis_index("s")
    rows_per_tile = n_pad // 16
    n_groups = chunks_per_tile // GRP
    rows_b = (rows0, rows1)
    src_g = (src_g0, src_g1)
    dst_g = (dst_g0, dst_g1)
    c_g = (c_g0, c_g1)
    gsem = (gsem0, gsem1)
    psem = (psem0, psem1)

    zero16 = jnp.zeros((LANES,), jnp.float32)

    def zrow(r, carry):
        for k in range(d // LANES):
            fout0[r, pl.ds(k * LANES, LANES)] = zero16
        ones_b[r, pl.ds(0, LANES)] = zero16
        return carry

    lax.fori_loop(0, CH, zrow, 0)
    for j in range(rows_per_tile // CH):
        r0 = si * rows_per_tile + j * CH
        pltpu.sync_copy(fout0, feat_acc.at[pl.ds(r0, CH)])
        pltpu.sync_copy(ones_b, deg_acc.at[pl.ds(r0, CH)])

    one16 = jnp.ones((LANES,), jnp.float32)

    def orow(r, carry):
        ones_b[r, pl.ds(0, LANES)] = one16
        return carry

    lax.fori_loop(0, CH, orow, 0)

    base = (ci * 16 + si) * chunks_per_tile
    pltpu.sync_copy(src_hbm.at[pl.ds(base, GRP)], src_g[0])
    pltpu.sync_copy(dst_hbm.at[pl.ds(base, GRP)], dst_g[0])
    pltpu.sync_copy(c_hbm.at[pl.ds(base, GRP)], c_g[0])
    plsc.subcore_barrier()

    pltpu.async_copy(x_hbm.at[src_g[0].at[0]], rows_b[0], gsem[0])

    def group_pair(gg, carry):
        for gb in range(2):
            g = gg * 2 + gb
            for cb in range(GRP):
                b = cb % 2
                t = g * GRP + cb

                if cb == 0:
                    @pl.when(g + 1 < n_groups)
                    def _prefetch():
                        nb = base + (g + 1) * GRP
                        pltpu.async_copy(src_hbm.at[pl.ds(nb, GRP)],
                                         src_g[1 - gb], psem[1 - gb])
                        pltpu.async_copy(dst_hbm.at[pl.ds(nb, GRP)],
                                         dst_g[1 - gb], psem[1 - gb])
                        pltpu.async_copy(c_hbm.at[pl.ds(nb, GRP)],
                                         c_g[1 - gb], psem[1 - gb])

                if cb < GRP - 1:
                    @pl.when(t + 1 < GRP * n_groups)
                    def _gather_next():
                        pltpu.async_copy(x_hbm.at[src_g[gb].at[cb + 1]],
                                         rows_b[1 - b], gsem[1 - b])
                else:
                    @pl.when(g + 1 < n_groups)
                    def _gather_next_group():
                        nbuf = 1 - gb
                        nb = base + (g + 1) * GRP
                        pltpu.make_async_copy(src_hbm.at[pl.ds(nb, GRP)],
                                              src_g[nbuf], psem[nbuf]).wait()
                        pltpu.make_async_copy(dst_hbm.at[pl.ds(nb, GRP)],
                                              dst_g[nbuf], psem[nbuf]).wait()
                        pltpu.make_async_copy(c_hbm.at[pl.ds(nb, GRP)],
                                              c_g[nbuf], psem[nbuf]).wait()
                        pltpu.async_copy(x_hbm.at[src_g[nbuf].at[0]],
                                         rows_b[1 - b], gsem[1 - b])

                pltpu.make_async_copy(x_hbm.at[src_g[gb].at[cb]],
                                      rows_b[b], gsem[b]).wait()

                @pl.when(t >= 1)
                def _drain():
                    pltpu.make_async_copy(
                        fout0, feat_acc.at[sidx.at[1 - b]], ssem0).wait()
                    pltpu.make_async_copy(
                        ones_b, deg_acc.at[sidx.at[1 - b]], ssem0).wait()

                for k in range(CH // LANES):
                    sl = pl.ds(k * LANES, LANES)
                    sidx[b, sl] = dst_g[gb][cb, sl]

                def srow(r, inner):
                    cbv = plsc.load_gather(
                        c_g[gb],
                        [jnp.full((LANES,), cb, jnp.int32),
                         jnp.full((LANES,), r, jnp.int32)])
                    for k in range(d // (2 * LANES)):
                        m = rows_b[b][r, pl.ds(k * 2 * LANES, 2 * LANES)]
                        lo, hi = plsc.unpack(
                            m, format=plsc.PackFormat.INTERLEAVED,
                            preferred_element_type=jnp.float32)
                        fout0[r, pl.ds(k * 2 * LANES, LANES)] = lo * cbv
                        fout0[r, pl.ds(k * 2 * LANES + LANES, LANES)] = \
                            hi * cbv
                    return inner

                lax.fori_loop(0, CH, srow, 0)

                pltpu.async_copy(fout0, feat_acc.at[sidx.at[b]],
                                 ssem0, add=True)
                pltpu.async_copy(ones_b, deg_acc.at[sidx.at[b]],
                                 ssem0, add=True)
        return carry

    lax.fori_loop(0, n_groups // 2, group_pair, 0)

    last_b = (chunks_per_tile - 1) % 2
    pltpu.make_async_copy(fout0, feat_acc.at[sidx.at[last_b]], ssem0).wait()
    pltpu.make_async_copy(ones_b, deg_acc.at[sidx.at[last_b]], ssem0).wait()
    plsc.subcore_barrier()

    r0 = si * rows_per_tile
    pltpu.sync_copy(feat_acc.at[pl.ds(r0, rows_per_tile)],
                    feat_hbm.at[pl.ds(ci * n_pad + r0, rows_per_tile)])
    pltpu.sync_copy(deg_acc.at[pl.ds(r0, rows_per_tile)],
                    deg_hbm.at[pl.ds(ci * n_pad + r0, rows_per_tile)])


def _combine_body(n_pad, d, f_ref, g_ref, o_ref):
    s = f_ref[:n_pad, :] + f_ref[n_pad:, :]
    deg = g_ref[:n_pad, :1] + g_ref[n_pad:, :1]
    o_ref[...] = s / (4.0 * (deg + 1e-9))


def kernel(x, edge_index, d_dists, d_phi, dists, sigma, phi, kappa):
    n, d = x.shape
    e = edge_index.shape[1]
    n_dist = dists.shape[0]
    n_phi = phi.shape[0]
    e_pad = -(-e // (N_TILES * CH)) * (N_TILES * CH)
    n_pad = -(-(n + 8) // (16 * CH)) * (16 * CH)
    chunks_per_tile = e_pad // (N_TILES * CH)

    params = jnp.concatenate(
        [dists, phi, sigma[None], kappa[None]]).astype(jnp.float32)
    rows_a = e // CH
    c2d = pl.pallas_call(
        functools.partial(_edge_weight_body, n_dist, n_phi),
        out_shape=jax.ShapeDtypeStruct((rows_a, CH), jnp.float32),
        in_specs=[
            pl.BlockSpec(memory_space=pltpu.SMEM),
            pl.BlockSpec(memory_space=pltpu.VMEM),
            pl.BlockSpec(memory_space=pltpu.VMEM),
        ],
        out_specs=pl.BlockSpec(memory_space=pltpu.VMEM),
    )(params, d_dists.reshape(rows_a, CH), d_phi.reshape(rows_a, CH))
    c = c2d.reshape(e)

    pad = e_pad - e
    n_chunks = e_pad // CH
    src_p = jnp.concatenate(
        [edge_index[0], jnp.zeros((pad,), jnp.int32)]).reshape(n_chunks, CH)
    dst_p = jnp.concatenate(
        [edge_index[1],
         jnp.full((pad,), n_pad - 8, jnp.int32)]).reshape(n_chunks, CH)
    c_p = jnp.concatenate(
        [c, jnp.zeros((pad,), jnp.float32)]).reshape(n_chunks, CH)

    mesh = plsc.VectorSubcoreMesh(core_axis_name="c", subcore_axis_name="s")
    sc_fn = pl.kernel(
        functools.partial(_sc_body, n_pad, chunks_per_tile, d),
        mesh=mesh,
        compiler_params=pltpu.CompilerParams(use_tc_tiling_on_sc=False,
                                             needs_layout_passes=False),
        out_type=[
            jax.ShapeDtypeStruct((2 * n_pad, d), jnp.float32),
            jax.ShapeDtypeStruct((2 * n_pad, LANES), jnp.float32),
        ],
        scratch_types=[
            pltpu.VMEM((GRP, CH), jnp.int32),
            pltpu.VMEM((GRP, CH), jnp.int32),
            pltpu.VMEM((GRP, CH), jnp.int32),
            pltpu.VMEM((GRP, CH), jnp.int32),
            pltpu.VMEM((GRP, CH), jnp.float32),
            pltpu.VMEM((GRP, CH), jnp.float32),
            pltpu.VMEM((CH, d), jnp.bfloat16),
            pltpu.VMEM((CH, d), jnp.bfloat16),
            pltpu.VMEM((CH, d), jnp.float32),
            pltpu.VMEM((2, CH), jnp.int32),
            pltpu.VMEM((CH, LANES), jnp.float32),
            pltpu.VMEM_SHARED((n_pad, d), jnp.float32),
            pltpu.VMEM_SHARED((n_pad, LANES), jnp.float32),
            pltpu.SemaphoreType.DMA,
            pltpu.SemaphoreType.DMA,
            pltpu.SemaphoreType.DMA,
            pltpu.SemaphoreType.DMA,
            pltpu.SemaphoreType.DMA,
        ],
    )
    perm = jnp.arange(d).reshape(d // 32, 2, 16).transpose(0, 2, 1).reshape(d)
    xb = x.astype(jnp.bfloat16)[:, perm]
    feat_p, deg_p = sc_fn(src_p, dst_p, c_p, xb)

    out_pad = pl.pallas_call(
        functools.partial(_combine_body, n_pad, d),
        out_shape=jax.ShapeDtypeStruct((n_pad, d), jnp.float32),
        in_specs=[pl.BlockSpec(memory_space=pltpu.VMEM),
                  pl.BlockSpec(memory_space=pltpu.VMEM)],
        out_specs=pl.BlockSpec(memory_space=pltpu.VMEM),
    )(feat_p, deg_p)
    return out_pad[:n, :]


# speedup vs baseline: 1.3861x; 1.0139x over previous
import functools

import jax
import jax.numpy as jnp
from jax import lax
from jax.experimental import pallas as pl
from jax.experimental.pallas import tpu as pltpu
from jax.experimental.pallas import tpu_sc as plsc

CH = 128
GRP = 4
LANES = 16
N_TILES = 32


def _edge_weight_body(n_dist, n_phi, params_ref, dd_ref, dp_ref, c_ref):
    dd = dd_ref[...]
    dp = dp_ref[...]
    sigma = params_ref[n_dist + n_phi]
    kappa = params_ref[n_dist + n_phi + 1]
    wd = jnp.zeros_like(dd)
    for k in range(n_dist):
        wd = wd + jnp.exp(-0.5 * ((dd - params_ref[k]) / sigma) ** 2)
    wp = jnp.zeros_like(dp)
    for j in range(n_phi):
        wp = wp + jnp.exp(kappa * jnp.cos(dp - params_ref[n_dist + j]))
    s = wd * wp
    c_ref[...] = s / (s + 1e-9)


def _sc_body(n_pad, chunks_per_tile, d,
             src_hbm, dst_hbm, c_hbm, x_hbm, feat_hbm, deg_hbm,
             src_g0, src_g1, dst_g0, dst_g1, c_g0, c_g1,
             rows0, rows1, fout0, sidx, ones_b, feat_acc, deg_acc,
             gsem0, gsem1, ssem0, psem0, psem1):
    ci = lax.axis_index("c")
    si = lax.axis_index("s")
    rows_per_tile = n_pad // 16
    n_groups = chunks_per_tile // GRP
    rows_b = (rows0, rows1)
    src_g = (src_g0, src_g1)
    dst_g = (dst_g0, dst_g1)
    c_g = (c_g0, c_g1)
    gsem = (gsem0, gsem1)
    psem = (psem0, psem1)

    zero16 = jnp.zeros((LANES,), jnp.float32)

    def zrow(r, carry):
        for k in range(d // LANES):
            fout0[r, pl.ds(k * LANES, LANES)] = zero16
        ones_b[r, pl.ds(0, LANES)] = zero16
        return carry

    lax.fori_loop(0, CH, zrow, 0)
    for j in range(rows_per_tile // CH):
        r0 = si * rows_per_tile + j * CH
        pltpu.sync_copy(fout0, feat_acc.at[pl.ds(r0, CH)])
        pltpu.sync_copy(ones_b, deg_acc.at[pl.ds(r0, CH)])

    one16 = jnp.ones((LANES,), jnp.float32)

    def orow(r, carry):
        ones_b[r, pl.ds(0, LANES)] = one16
        return carry

    lax.fori_loop(0, CH, orow, 0)

    base = (ci * 16 + si) * chunks_per_tile
    pltpu.sync_copy(src_hbm.at[pl.ds(base, GRP)], src_g[0])
    pltpu.sync_copy(dst_hbm.at[pl.ds(base, GRP)], dst_g[0])
    pltpu.sync_copy(c_hbm.at[pl.ds(base, GRP)], c_g[0])
    plsc.subcore_barrier()

    pltpu.async_copy(x_hbm.at[src_g[0].at[0]], rows_b[0], gsem[0])

    def group_pair(gg, carry):
        for gb in range(2):
            g = gg * 2 + gb
            for cb in range(GRP):
                b = cb % 2
                t = g * GRP + cb

                if cb == 0:
                    @pl.when(g + 1 < n_groups)
                    def _prefetch():
                        nb = base + (g + 1) * GRP
                        pltpu.async_copy(src_hbm.at[pl.ds(nb, GRP)],
                                         src_g[1 - gb], psem[1 - gb])
                        pltpu.async_copy(dst_hbm.at[pl.ds(nb, GRP)],
                                         dst_g[1 - gb], psem[1 - gb])
                        pltpu.async_copy(c_hbm.at[pl.ds(nb, GRP)],
                                         c_g[1 - gb], psem[1 - gb])

                if cb < GRP - 1:
                    @pl.when(t + 1 < GRP * n_groups)
                    def _gather_next():
                        pltpu.async_copy(x_hbm.at[src_g[gb].at[cb + 1]],
                                         rows_b[1 - b], gsem[1 - b])
                else:
                    @pl.when(g + 1 < n_groups)
                    def _gather_next_group():
                        nbuf = 1 - gb
                        nb = base + (g + 1) * GRP
                        pltpu.make_async_copy(src_hbm.at[pl.ds(nb, GRP)],
                                              src_g[nbuf], psem[nbuf]).wait()
                        pltpu.make_async_copy(dst_hbm.at[pl.ds(nb, GRP)],
                                              dst_g[nbuf], psem[nbuf]).wait()
                        pltpu.make_async_copy(c_hbm.at[pl.ds(nb, GRP)],
                                              c_g[nbuf], psem[nbuf]).wait()
                        pltpu.async_copy(x_hbm.at[src_g[nbuf].at[0]],
                                         rows_b[1 - b], gsem[1 - b])

                pltpu.make_async_copy(x_hbm.at[src_g[gb].at[cb]],
                                      rows_b[b], gsem[b]).wait()

                @pl.when(t >= 1)
                def _drain():
                    pltpu.make_async_copy(
                        fout0, feat_acc.at[sidx.at[1 - b]], ssem0).wait()
                    pltpu.make_async_copy(
                        ones_b, deg_acc.at[sidx.at[1 - b]], ssem0).wait()

                for k in range(CH // LANES):
                    sl = pl.ds(k * LANES, LANES)
                    sidx[b, sl] = dst_g[gb][cb, sl]

                def srow(rr, inner):
                    for u in range(2):
                        r = rr * 2 + u
                        cbv = plsc.load_gather(
                            c_g[gb],
                            [jnp.full((LANES,), cb, jnp.int32),
                             jnp.full((LANES,), r, jnp.int32)])
                        for k in range(d // (2 * LANES)):
                            m = rows_b[b][r, pl.ds(k * 2 * LANES,
                                                   2 * LANES)]
                            lo, hi = plsc.unpack(
                                m, format=plsc.PackFormat.INTERLEAVED,
                                preferred_element_type=jnp.float32)
                            fout0[r, pl.ds(k * 2 * LANES, LANES)] = \
                                lo * cbv
                            fout0[r, pl.ds(k * 2 * LANES + LANES,
                                           LANES)] = hi * cbv
                    return inner

                lax.fori_loop(0, CH // 2, srow, 0)

                pltpu.async_copy(fout0, feat_acc.at[sidx.at[b]],
                                 ssem0, add=True)
                pltpu.async_copy(ones_b, deg_acc.at[sidx.at[b]],
                                 ssem0, add=True)
        return carry

    lax.fori_loop(0, n_groups // 2, group_pair, 0)

    last_b = (chunks_per_tile - 1) % 2
    pltpu.make_async_copy(fout0, feat_acc.at[sidx.at[last_b]], ssem0).wait()
    pltpu.make_async_copy(ones_b, deg_acc.at[sidx.at[last_b]], ssem0).wait()
    plsc.subcore_barrier()

    r0 = si * rows_per_tile
    pltpu.sync_copy(feat_acc.at[pl.ds(r0, rows_per_tile)],
                    feat_hbm.at[pl.ds(ci * n_pad + r0, rows_per_tile)])
    pltpu.sync_copy(deg_acc.at[pl.ds(r0, rows_per_tile)],
                    deg_hbm.at[pl.ds(ci * n_pad + r0, rows_per_tile)])


def _combine_body(n, n_pad, d, f_ref, g_ref, o_ref):
    s = f_ref[:n, :] + f_ref[n_pad:n_pad + n, :]
    deg = g_ref[:n, :1] + g_ref[n_pad:n_pad + n, :1]
    o_ref[...] = s / (4.0 * (deg + 1e-9))


def kernel(x, edge_index, d_dists, d_phi, dists, sigma, phi, kappa):
    n, d = x.shape
    e = edge_index.shape[1]
    n_dist = dists.shape[0]
    n_phi = phi.shape[0]
    e_pad = -(-e // (N_TILES * CH)) * (N_TILES * CH)
    n_pad = -(-(n + 8) // (16 * CH)) * (16 * CH)
    chunks_per_tile = e_pad // (N_TILES * CH)

    params = jnp.concatenate(
        [dists, phi, sigma[None], kappa[None]]).astype(jnp.float32)
    rows_a = e // CH
    c2d = pl.pallas_call(
        functools.partial(_edge_weight_body, n_dist, n_phi),
        out_shape=jax.ShapeDtypeStruct((rows_a, CH), jnp.float32),
        in_specs=[
            pl.BlockSpec(memory_space=pltpu.SMEM),
            pl.BlockSpec(memory_space=pltpu.VMEM),
            pl.BlockSpec(memory_space=pltpu.VMEM),
        ],
        out_specs=pl.BlockSpec(memory_space=pltpu.VMEM),
    )(params, d_dists.reshape(rows_a, CH), d_phi.reshape(rows_a, CH))
    c = c2d.reshape(e)

    pad = e_pad - e
    n_chunks = e_pad // CH
    src_p = jnp.concatenate(
        [edge_index[0], jnp.zeros((pad,), jnp.int32)]).reshape(n_chunks, CH)
    dst_p = jnp.concatenate(
        [edge_index[1],
         jnp.full((pad,), n_pad - 8, jnp.int32)]).reshape(n_chunks, CH)
    c_p = jnp.concatenate(
        [c, jnp.zeros((pad,), jnp.float32)]).reshape(n_chunks, CH)

    mesh = plsc.VectorSubcoreMesh(core_axis_name="c", subcore_axis_name="s")
    sc_fn = pl.kernel(
        functools.partial(_sc_body, n_pad, chunks_per_tile, d),
        mesh=mesh,
        compiler_params=pltpu.CompilerParams(use_tc_tiling_on_sc=False,
                                             needs_layout_passes=False),
        out_type=[
            jax.ShapeDtypeStruct((2 * n_pad, d), jnp.float32),
            jax.ShapeDtypeStruct((2 * n_pad, LANES), jnp.float32),
        ],
        scratch_types=[
            pltpu.VMEM((GRP, CH), jnp.int32),
            pltpu.VMEM((GRP, CH), jnp.int32),
            pltpu.VMEM((GRP, CH), jnp.int32),
            pltpu.VMEM((GRP, CH), jnp.int32),
            pltpu.VMEM((GRP, CH), jnp.float32),
            pltpu.VMEM((GRP, CH), jnp.float32),
            pltpu.VMEM((CH, d), jnp.bfloat16),
            pltpu.VMEM((CH, d), jnp.bfloat16),
            pltpu.VMEM((CH, d), jnp.float32),
            pltpu.VMEM((2, CH), jnp.int32),
            pltpu.VMEM((CH, LANES), jnp.float32),
            pltpu.VMEM_SHARED((n_pad, d), jnp.float32),
            pltpu.VMEM_SHARED((n_pad, LANES), jnp.float32),
            pltpu.SemaphoreType.DMA,
            pltpu.SemaphoreType.DMA,
            pltpu.SemaphoreType.DMA,
            pltpu.SemaphoreType.DMA,
            pltpu.SemaphoreType.DMA,
        ],
    )
    perm = jnp.arange(d).reshape(d // 32, 2, 16).transpose(0, 2, 1).reshape(d)
    xb = x.astype(jnp.bfloat16)[:, perm]
    feat_p, deg_p = sc_fn(src_p, dst_p, c_p, xb)

    return pl.pallas_call(
        functools.partial(_combine_body, n, n_pad, d),
        out_shape=jax.ShapeDtypeStruct((n, d), jnp.float32),
        in_specs=[pl.BlockSpec(memory_space=pltpu.VMEM),
                  pl.BlockSpec(memory_space=pltpu.VMEM)],
        out_specs=pl.BlockSpec(memory_space=pltpu.VMEM),
    )(feat_p, deg_p)
